# baseline (speedup 1.0000x reference)
import functools

import jax
import jax.numpy as jnp
from jax.experimental import pallas as pl
from jax.experimental.pallas import tpu as pltpu

_N_EXPERTS = 64
_TOP_K = 8


def _router_block(x_ref, w_ref, b_ref, prob_ref, idx_ref, z_ref):
    _INT_MIN = jnp.int32(-2147483648)

    bias_col = jnp.transpose(b_ref[...], (1, 0))
    logits = jax.lax.dot_general(
        w_ref[...], x_ref[0],
        (((1,), (1,)), ((), ())),
        preferred_element_type=jnp.float32,
    ) + bias_col

    t = logits.shape[1]

    bits = jax.lax.bitcast_convert_type(logits, jnp.int32)
    key = jnp.where(bits >= 0, bits, _INT_MIN - bits)
    e_iota = jax.lax.broadcasted_iota(jnp.int32, (_N_EXPERTS, t), 0)
    key = (key & jnp.int32(-64)) | (jnp.int32(63) - e_iota)

    maxes = []
    for _ in range(_TOP_K):
        m = jnp.max(key, axis=0, keepdims=True)
        maxes.append(m)
        key = jnp.where(key == m, _INT_MIN, key)
    kstack = jnp.concatenate(maxes, axis=0)

    idx = jnp.int32(63) - (kstack & jnp.int32(63))
    sbits = jnp.where(kstack >= 0, kstack, _INT_MIN - kstack)
    scores = jax.lax.bitcast_convert_type(sbits, jnp.float32)

    m0 = scores[0:1, :]
    p = jnp.exp(scores - m0)
    prob_ref[0] = p / jnp.sum(p, axis=0, keepdims=True)
    idx_ref[0] = idx

    se = jnp.sum(jnp.exp(logits - m0), axis=0, keepdims=True)
    lse = m0 + jnp.log(se)

    @pl.when((pl.program_id(0) == 0) & (pl.program_id(1) == 0))
    def _():
        z_ref[...] = jnp.zeros((1, 1), jnp.float32)

    z_ref[...] += jnp.sum(lse * lse, axis=1, keepdims=True)


@functools.partial(jax.jit, static_argnames=())
def kernel(x, W, expert_bias):
    b, s, d = x.shape
    block_t = 2048
    grid = (b, s // block_t)

    prob, idx, zsum = pl.pallas_call(
        _router_block,
        grid=grid,
        in_specs=[
            pl.BlockSpec((1, block_t, d), lambda i, j: (i, j, 0)),
            pl.BlockSpec((_N_EXPERTS, d), lambda i, j: (0, 0)),
            pl.BlockSpec((1, _N_EXPERTS), lambda i, j: (0, 0)),
        ],
        out_specs=[
            pl.BlockSpec((1, _TOP_K, block_t), lambda i, j: (i, 0, j)),
            pl.BlockSpec((1, _TOP_K, block_t), lambda i, j: (i, 0, j)),
            pl.BlockSpec((1, 1), lambda i, j: (0, 0)),
        ],
        out_shape=[
            jax.ShapeDtypeStruct((b, _TOP_K, s), jnp.float32),
            jax.ShapeDtypeStruct((b, _TOP_K, s), jnp.int32),
            jax.ShapeDtypeStruct((1, 1), jnp.float32),
        ],
        compiler_params=pltpu.CompilerParams(
            dimension_semantics=("arbitrary", "arbitrary"),
        ),
    )(x, W, expert_bias.reshape(1, _N_EXPERTS))

    return (prob.transpose(0, 2, 1),
            idx.transpose(0, 2, 1),
            zsum[0, 0] / jnp.float32(b * s))


# speedup vs baseline: 1.0990x; 1.0990x over previous
import functools

import jax
import jax.numpy as jnp
from jax.experimental import pallas as pl
from jax.experimental.pallas import tpu as pltpu

_N_EXPERTS = 64
_TOP_K = 8


def _router_block(x_ref, w_ref, b_ref, prob_ref, idx_ref, z_ref):
    _INT_MIN = jnp.int32(-2147483648)

    bias_col = jnp.transpose(b_ref[...], (1, 0))
    logits = jax.lax.dot_general(
        w_ref[...], x_ref[0],
        (((1,), (1,)), ((), ())),
        preferred_element_type=jnp.float32,
    ) + bias_col

    t = logits.shape[1]

    bits = jax.lax.bitcast_convert_type(logits, jnp.int32)
    key = jnp.where(bits >= 0, bits, _INT_MIN - bits)
    e_iota = jax.lax.broadcasted_iota(jnp.int32, (_N_EXPERTS, t), 0)
    key = (key & jnp.int32(-64)) | (jnp.int32(63) - e_iota)

    maxes = []
    for _ in range(_TOP_K):
        m = jnp.max(key, axis=0, keepdims=True)
        maxes.append(m)
        key = jnp.where(key == m, _INT_MIN, key)
    kstack = jnp.concatenate(maxes, axis=0)

    idx = jnp.int32(63) - (kstack & jnp.int32(63))
    sbits = jnp.where(kstack >= 0, kstack, _INT_MIN - kstack)
    scores = jax.lax.bitcast_convert_type(sbits, jnp.float32)

    m0 = scores[0:1, :]
    p = jnp.exp(scores - m0)
    prob_ref[0] = p / jnp.sum(p, axis=0, keepdims=True)
    idx_ref[0] = idx

    se = jnp.sum(jnp.exp(logits - m0), axis=0, keepdims=True)
    lse = m0 + jnp.log(se)

    @pl.when((pl.program_id(0) == 0) & (pl.program_id(1) == 0))
    def _():
        z_ref[...] = jnp.zeros((1, 1), jnp.float32)

    z_ref[...] += jnp.sum(lse * lse, axis=1, keepdims=True)


@functools.partial(jax.jit, static_argnames=())
def kernel(x, W, expert_bias):
    b, s, d = x.shape
    block_t = 4096
    grid = (b, s // block_t)

    prob, idx, zsum = pl.pallas_call(
        _router_block,
        grid=grid,
        in_specs=[
            pl.BlockSpec((1, block_t, d), lambda i, j: (i, j, 0)),
            pl.BlockSpec((_N_EXPERTS, d), lambda i, j: (0, 0)),
            pl.BlockSpec((1, _N_EXPERTS), lambda i, j: (0, 0)),
        ],
        out_specs=[
            pl.BlockSpec((1, _TOP_K, block_t), lambda i, j: (i, 0, j)),
            pl.BlockSpec((1, _TOP_K, block_t), lambda i, j: (i, 0, j)),
            pl.BlockSpec((1, 1), lambda i, j: (0, 0)),
        ],
        out_shape=[
            jax.ShapeDtypeStruct((b, _TOP_K, s), jnp.float32),
            jax.ShapeDtypeStruct((b, _TOP_K, s), jnp.int32),
            jax.ShapeDtypeStruct((1, 1), jnp.float32),
        ],
        compiler_params=pltpu.CompilerParams(
            dimension_semantics=("arbitrary", "arbitrary"),
        ),
    )(x, W, expert_bias.reshape(1, _N_EXPERTS))

    return (prob.transpose(0, 2, 1),
            idx.transpose(0, 2, 1),
            zsum[0, 0] / jnp.float32(b * s))
